# Initial kernel scaffold; baseline (speedup 1.0000x reference)
import functools

import jax
import jax.numpy as jnp
from jax import lax
from jax.experimental import pallas as pl
from jax.experimental.pallas import tpu as pltpu
from jax.experimental.pallas import tpu_sc as plsc

N = 10000
D = 128
E = 320000
NC = 2
NS = 16
NW = NC * NS
EPW = E // NW
C = 80
NCHUNK = EPW // C
RPT = N // NS
ZR = 125
DW = 16


def _zero_fill(buf, nrows, width):
    @pl.loop(0, nrows)
    def _(r):
        @pl.loop(0, width, step=16)
        def _(j):
            buf[r, pl.ds(j, 16)] = jnp.zeros((16,), jnp.float32)


def _sc_degree(dstw):
    mesh = plsc.VectorSubcoreMesh(core_axis_name="c", subcore_axis_name="s")

    @functools.partial(
        pl.kernel,
        out_type=jax.ShapeDtypeStruct((NC, N, DW), jnp.float32),
        mesh=mesh,
        scratch_types=[
            pltpu.VMEM((NCHUNK, C), jnp.int32),
            pltpu.VMEM((C, DW), jnp.float32),
            pltpu.VMEM((ZR, DW), jnp.float32),
            pltpu.VMEM_SHARED((N, DW), jnp.float32),
        ],
    )
    def deg_kernel(dst_hbm, out_hbm, idxb, onesb, zb, accum):
        cid = lax.axis_index("c")
        sid = lax.axis_index("s")
        wid = sid * NC + cid
        _zero_fill(zb, ZR, DW)

        @pl.loop(0, C)
        def _(r):
            onesb[r, :] = jnp.ones((16,), jnp.float32)

        @pl.loop(0, RPT // ZR)
        def _(k):
            pltpu.sync_copy(zb, accum.at[pl.ds(sid * RPT + k * ZR, ZR)])

        plsc.subcore_barrier()
        pltpu.sync_copy(dst_hbm.at[wid], idxb)

        @pl.loop(0, NCHUNK)
        def _(i):
            pltpu.sync_copy(onesb, accum.at[idxb.at[i]], add=True)

        plsc.subcore_barrier()
        pltpu.sync_copy(accum.at[pl.ds(sid * RPT, RPT)],
                        out_hbm.at[cid, pl.ds(sid * RPT, RPT)])

    return deg_kernel(dstw)


def _sc_message(values, srcw, dstw):
    mesh = plsc.VectorSubcoreMesh(core_axis_name="c", subcore_axis_name="s")

    @functools.partial(
        pl.kernel,
        out_type=jax.ShapeDtypeStruct((NC, N, D), jnp.float32),
        mesh=mesh,
        scratch_types=[
            pltpu.VMEM((NCHUNK, C), jnp.int32),
            pltpu.VMEM((NCHUNK, C), jnp.int32),
            pltpu.VMEM((C, D), jnp.float32),
            pltpu.VMEM((ZR, D), jnp.float32),
            pltpu.VMEM_SHARED((N, D), jnp.float32),
            pltpu.SemaphoreType.DMA,
        ],
    )
    def msg_kernel(val_hbm, src_hbm, dst_hbm, out_hbm, srcb, dstb, rows, zb,
                   accum, sem):
        cid = lax.axis_index("c")
        sid = lax.axis_index("s")
        wid = sid * NC + cid
        _zero_fill(zb, ZR, D)

        @pl.loop(0, RPT // ZR)
        def _(k):
            pltpu.sync_copy(zb, accum.at[pl.ds(sid * RPT + k * ZR, ZR)])

        plsc.subcore_barrier()
        pltpu.sync_copy(src_hbm.at[wid], srcb)
        pltpu.sync_copy(dst_hbm.at[wid], dstb)

        @pl.loop(0, NCHUNK)
        def _(i):
            pltpu.async_copy(val_hbm.at[srcb.at[i]], rows, sem).wait()
            pltpu.sync_copy(rows, accum.at[dstb.at[i]], add=True)

        plsc.subcore_barrier()
        pltpu.sync_copy(accum.at[pl.ds(sid * RPT, RPT)],
                        out_hbm.at[cid, pl.ds(sid * RPT, RPT)])

    return msg_kernel(values, srcw, dstw)


_BR = 1000


def _deg_dis(deg_ref):
    deg = 1.0 + deg_ref[0, :, 0:1] + deg_ref[1, :, 0:1]
    return lax.rsqrt(deg)


def _tc_hws1(degp, x, W_in, b_in, W1):
    def body(deg_ref, x_ref, win_ref, bin_ref, w1_ref, o_ref):
        h0 = jnp.dot(x_ref[...], win_ref[...],
                     preferred_element_type=jnp.float32) + bin_ref[...]
        hw1 = jnp.dot(h0, w1_ref[...], preferred_element_type=jnp.float32)
        o_ref[...] = hw1 * _deg_dis(deg_ref)

    return pl.pallas_call(
        body,
        grid=(N // _BR,),
        in_specs=[
            pl.BlockSpec((NC, _BR, DW), lambda i: (0, i, 0)),
            pl.BlockSpec((_BR, D), lambda i: (i, 0)),
            pl.BlockSpec((D, D), lambda i: (0, 0)),
            pl.BlockSpec((1, D), lambda i: (0, 0)),
            pl.BlockSpec((D, D), lambda i: (0, 0)),
        ],
        out_specs=pl.BlockSpec((_BR, D), lambda i: (i, 0)),
        out_shape=jax.ShapeDtypeStruct((N, D), jnp.float32),
    )(degp, x, W_in, b_in.reshape(1, D), W1)


def _tc_mid(degp, mp, hws1, b1, W2):
    def body(deg_ref, mp_ref, hws1_ref, b1_ref, w2_ref, o_ref):
        dis = _deg_dis(deg_ref)
        s = mp_ref[0] + mp_ref[1] + hws1_ref[...]
        h1 = jnp.maximum(dis * s + b1_ref[...], 0.0)
        hw2 = jnp.dot(h1, w2_ref[...], preferred_element_type=jnp.float32)
        o_ref[...] = hw2 * dis

    return pl.pallas_call(
        body,
        grid=(N // _BR,),
        in_specs=[
            pl.BlockSpec((NC, _BR, DW), lambda i: (0, i, 0)),
            pl.BlockSpec((NC, _BR, D), lambda i: (0, i, 0)),
            pl.BlockSpec((_BR, D), lambda i: (i, 0)),
            pl.BlockSpec((1, D), lambda i: (0, 0)),
            pl.BlockSpec((D, D), lambda i: (0, 0)),
        ],
        out_specs=pl.BlockSpec((_BR, D), lambda i: (i, 0)),
        out_shape=jax.ShapeDtypeStruct((N, D), jnp.float32),
    )(degp, mp, hws1, b1.reshape(1, D), W2)


def _tc_final(degp, mp, hws2, b2):
    def body(deg_ref, mp_ref, hws2_ref, b2_ref, o_ref):
        dis = _deg_dis(deg_ref)
        s = mp_ref[0] + mp_ref[1] + hws2_ref[...]
        o_ref[...] = dis * s + b2_ref[...]

    return pl.pallas_call(
        body,
        grid=(N // _BR,),
        in_specs=[
            pl.BlockSpec((NC, _BR, DW), lambda i: (0, i, 0)),
            pl.BlockSpec((NC, _BR, D), lambda i: (0, i, 0)),
            pl.BlockSpec((_BR, D), lambda i: (i, 0)),
            pl.BlockSpec((1, D), lambda i: (0, 0)),
        ],
        out_specs=pl.BlockSpec((_BR, D), lambda i: (i, 0)),
        out_shape=jax.ShapeDtypeStruct((N, D), jnp.float32),
    )(degp, mp, hws2, b2.reshape(1, D))


def kernel(x, edge_index, W_in, b_in, W1, b1, W2, b2):
    ei = edge_index.astype(jnp.int32)
    srcw = ei[0].reshape(NW, NCHUNK, C)
    dstw = ei[1].reshape(NW, NCHUNK, C)
    degp = _sc_degree(dstw)
    hws1 = _tc_hws1(degp, x, W_in, b_in, W1)
    mp1 = _sc_message(hws1, srcw, dstw)
    hws2 = _tc_mid(degp, mp1, hws1, b1, W2)
    mp2 = _sc_message(hws2, srcw, dstw)
    return _tc_final(degp, mp2, hws2, b2)


# speedup vs baseline: 19.5932x; 19.5932x over previous
import functools

import jax
import jax.numpy as jnp
from jax import lax
from jax.experimental import pallas as pl
from jax.experimental.pallas import tpu as pltpu
from jax.experimental.pallas import tpu_sc as plsc

N = 10000
D = 128
E = 320000
NC = 2
NS = 16
NW = NC * NS
EPW = E // NW
C = 80
NCHUNK = EPW // C
RPT = 624
TAIL = N - NS * RPT
ZB = 48
DW = 16


def _zero_fill(buf, nrows, width):
    @pl.loop(0, nrows)
    def _(r):
        @pl.loop(0, width, step=16)
        def _(j):
            buf[r, pl.ds(j, 16)] = jnp.zeros((16,), jnp.float32)


def _zero_accum(sid, zb, accum):
    @pl.loop(0, RPT // ZB)
    def _(k):
        pltpu.sync_copy(zb, accum.at[pl.ds(sid * RPT + k * ZB, ZB)])

    @pl.when(sid == NS - 1)
    def _():
        pltpu.sync_copy(zb.at[pl.ds(0, TAIL)], accum.at[pl.ds(NS * RPT, TAIL)])


def _drain_accum(cid, sid, accum, out_hbm):
    pltpu.sync_copy(accum.at[pl.ds(sid * RPT, RPT)],
                    out_hbm.at[cid, pl.ds(sid * RPT, RPT)])

    @pl.when(sid == NS - 1)
    def _():
        pltpu.sync_copy(accum.at[pl.ds(NS * RPT, TAIL)],
                        out_hbm.at[cid, pl.ds(NS * RPT, TAIL)])


def _sc_degree(dstw):
    mesh = plsc.VectorSubcoreMesh(core_axis_name="c", subcore_axis_name="s")

    @functools.partial(
        pl.kernel,
        out_type=jax.ShapeDtypeStruct((NC, N, DW), jnp.float32),
        mesh=mesh,
        scratch_types=[
            pltpu.VMEM((NCHUNK, C), jnp.int32),
            pltpu.VMEM((C, DW), jnp.float32),
            pltpu.VMEM((ZB, DW), jnp.float32),
            pltpu.VMEM_SHARED((N, DW), jnp.float32),
        ],
    )
    def deg_kernel(dst_hbm, out_hbm, idxb, onesb, zb, accum):
        cid = lax.axis_index("c")
        sid = lax.axis_index("s")
        wid = sid * NC + cid
        _zero_fill(zb, ZB, DW)

        @pl.loop(0, C)
        def _(r):
            onesb[r, :] = jnp.ones((16,), jnp.float32)

        _zero_accum(sid, zb, accum)
        plsc.subcore_barrier()
        pltpu.sync_copy(dst_hbm.at[wid], idxb)

        @pl.loop(0, NCHUNK)
        def _(i):
            pltpu.sync_copy(onesb, accum.at[idxb.at[i]], add=True)

        plsc.subcore_barrier()
        _drain_accum(cid, sid, accum, out_hbm)

    return deg_kernel(dstw)


def _sc_message(values, srcw, dstw):
    mesh = plsc.VectorSubcoreMesh(core_axis_name="c", subcore_axis_name="s")

    @functools.partial(
        pl.kernel,
        out_type=jax.ShapeDtypeStruct((NC, N, D), jnp.float32),
        mesh=mesh,
        scratch_types=[
            pltpu.VMEM((NCHUNK, C), jnp.int32),
            pltpu.VMEM((NCHUNK, C), jnp.int32),
            pltpu.VMEM((C, D), jnp.float32),
            pltpu.VMEM((ZB, D), jnp.float32),
            pltpu.VMEM_SHARED((N, D), jnp.float32),
            pltpu.SemaphoreType.DMA,
        ],
    )
    def msg_kernel(val_hbm, src_hbm, dst_hbm, out_hbm, srcb, dstb, rows, zb,
                   accum, sem):
        cid = lax.axis_index("c")
        sid = lax.axis_index("s")
        wid = sid * NC + cid
        _zero_fill(zb, ZB, D)
        _zero_accum(sid, zb, accum)
        plsc.subcore_barrier()
        pltpu.sync_copy(src_hbm.at[wid], srcb)
        pltpu.sync_copy(dst_hbm.at[wid], dstb)

        @pl.loop(0, NCHUNK)
        def _(i):
            pltpu.async_copy(val_hbm.at[srcb.at[i]], rows, sem).wait()
            pltpu.sync_copy(rows, accum.at[dstb.at[i]], add=True)

        plsc.subcore_barrier()
        _drain_accum(cid, sid, accum, out_hbm)

    return msg_kernel(values, srcw, dstw)


_BR = 1000


def _deg_dis(deg_ref):
    deg = 1.0 + deg_ref[0, :, 0:1] + deg_ref[1, :, 0:1]
    return lax.rsqrt(deg)


def _tc_hws1(degp, x, W_in, b_in, W1):
    def body(deg_ref, x_ref, win_ref, bin_ref, w1_ref, o_ref):
        h0 = jnp.dot(x_ref[...], win_ref[...],
                     preferred_element_type=jnp.float32) + bin_ref[...]
        hw1 = jnp.dot(h0, w1_ref[...], preferred_element_type=jnp.float32)
        o_ref[...] = hw1 * _deg_dis(deg_ref)

    return pl.pallas_call(
        body,
        grid=(N // _BR,),
        in_specs=[
            pl.BlockSpec((NC, _BR, DW), lambda i: (0, i, 0)),
            pl.BlockSpec((_BR, D), lambda i: (i, 0)),
            pl.BlockSpec((D, D), lambda i: (0, 0)),
            pl.BlockSpec((1, D), lambda i: (0, 0)),
            pl.BlockSpec((D, D), lambda i: (0, 0)),
        ],
        out_specs=pl.BlockSpec((_BR, D), lambda i: (i, 0)),
        out_shape=jax.ShapeDtypeStruct((N, D), jnp.float32),
    )(degp, x, W_in, b_in.reshape(1, D), W1)


def _tc_mid(degp, mp, hws1, b1, W2):
    def body(deg_ref, mp_ref, hws1_ref, b1_ref, w2_ref, o_ref):
        dis = _deg_dis(deg_ref)
        s = mp_ref[0] + mp_ref[1] + hws1_ref[...]
        h1 = jnp.maximum(dis * s + b1_ref[...], 0.0)
        hw2 = jnp.dot(h1, w2_ref[...], preferred_element_type=jnp.float32)
        o_ref[...] = hw2 * dis

    return pl.pallas_call(
        body,
        grid=(N // _BR,),
        in_specs=[
            pl.BlockSpec((NC, _BR, DW), lambda i: (0, i, 0)),
            pl.BlockSpec((NC, _BR, D), lambda i: (0, i, 0)),
            pl.BlockSpec((_BR, D), lambda i: (i, 0)),
            pl.BlockSpec((1, D), lambda i: (0, 0)),
            pl.BlockSpec((D, D), lambda i: (0, 0)),
        ],
        out_specs=pl.BlockSpec((_BR, D), lambda i: (i, 0)),
        out_shape=jax.ShapeDtypeStruct((N, D), jnp.float32),
    )(degp, mp, hws1, b1.reshape(1, D), W2)


def _tc_final(degp, mp, hws2, b2):
    def body(deg_ref, mp_ref, hws2_ref, b2_ref, o_ref):
        dis = _deg_dis(deg_ref)
        s = mp_ref[0] + mp_ref[1] + hws2_ref[...]
        o_ref[...] = dis * s + b2_ref[...]

    return pl.pallas_call(
        body,
        grid=(N // _BR,),
        in_specs=[
            pl.BlockSpec((NC, _BR, DW), lambda i: (0, i, 0)),
            pl.BlockSpec((NC, _BR, D), lambda i: (0, i, 0)),
            pl.BlockSpec((_BR, D), lambda i: (i, 0)),
            pl.BlockSpec((1, D), lambda i: (0, 0)),
        ],
        out_specs=pl.BlockSpec((_BR, D), lambda i: (i, 0)),
        out_shape=jax.ShapeDtypeStruct((N, D), jnp.float32),
    )(degp, mp, hws2, b2.reshape(1, D))


def kernel(x, edge_index, W_in, b_in, W1, b1, W2, b2):
    ei = edge_index.astype(jnp.int32)
    srcw = ei[0].reshape(NW, NCHUNK, C)
    dstw = ei[1].reshape(NW, NCHUNK, C)
    degp = _sc_degree(dstw)
    hws1 = _tc_hws1(degp, x, W_in, b_in, W1)
    mp1 = _sc_message(hws1, srcw, dstw)
    hws2 = _tc_mid(degp, mp1, hws1, b1, W2)
    mp2 = _sc_message(hws2, srcw, dstw)
    return _tc_final(degp, mp2, hws2, b2)


# speedup vs baseline: 25.8943x; 1.3216x over previous
import functools

import jax
import jax.numpy as jnp
from jax import lax
from jax.experimental import pallas as pl
from jax.experimental.pallas import tpu as pltpu
from jax.experimental.pallas import tpu_sc as plsc

N = 10000
D = 128
DH = D // 2
E = 320000
NC = 2
NS = 16
EPS = E // NS
C = 125
NCHUNK = EPS // C
NCHD = NCHUNK // NC
RPT = 624
TAIL = N - NS * RPT
ZB = 16
DW = 16


def _zero_fill(buf, nrows, width):
    @pl.loop(0, nrows)
    def _(r):
        @pl.loop(0, width, step=16)
        def _(j):
            buf[r, pl.ds(j, 16)] = jnp.zeros((16,), jnp.float32)


def _zero_accum(sid, zb, accum):
    @pl.loop(0, RPT // ZB)
    def _(k):
        pltpu.sync_copy(zb, accum.at[pl.ds(sid * RPT + k * ZB, ZB)])

    @pl.when(sid == NS - 1)
    def _():
        pltpu.sync_copy(zb.at[pl.ds(0, TAIL)], accum.at[pl.ds(NS * RPT, TAIL)])


def _drain_accum(cid, sid, accum, out_hbm):
    pltpu.sync_copy(accum.at[pl.ds(sid * RPT, RPT)],
                    out_hbm.at[cid, pl.ds(sid * RPT, RPT)])

    @pl.when(sid == NS - 1)
    def _():
        pltpu.sync_copy(accum.at[pl.ds(NS * RPT, TAIL)],
                        out_hbm.at[cid, pl.ds(NS * RPT, TAIL)])


def _sc_degree(dstw):
    mesh = plsc.VectorSubcoreMesh(core_axis_name="c", subcore_axis_name="s")

    @functools.partial(
        pl.kernel,
        out_type=jax.ShapeDtypeStruct((NC, N, DW), jnp.float32),
        mesh=mesh,
        scratch_types=[
            pltpu.VMEM((NCHD, C), jnp.int32),
            pltpu.VMEM((C, DW), jnp.float32),
            pltpu.VMEM((ZB, DW), jnp.float32),
            pltpu.VMEM_SHARED((N, DW), jnp.float32),
        ],
    )
    def deg_kernel(dst_hbm, out_hbm, idxb, onesb, zb, accum):
        cid = lax.axis_index("c")
        sid = lax.axis_index("s")
        _zero_fill(zb, ZB, DW)

        @pl.loop(0, C)
        def _(r):
            onesb[r, :] = jnp.ones((16,), jnp.float32)

        _zero_accum(sid, zb, accum)
        plsc.subcore_barrier()
        pltpu.sync_copy(dst_hbm.at[sid, pl.ds(cid * NCHD, NCHD)], idxb)

        @pl.loop(0, NCHD)
        def _(i):
            pltpu.sync_copy(onesb, accum.at[idxb.at[i]], add=True)

        plsc.subcore_barrier()
        _drain_accum(cid, sid, accum, out_hbm)

    return deg_kernel(dstw)


def _sc_message(values, srcw, dstw):
    mesh = plsc.VectorSubcoreMesh(core_axis_name="c", subcore_axis_name="s")

    @functools.partial(
        pl.kernel,
        out_type=jax.ShapeDtypeStruct((NC, N, DH), jnp.float32),
        mesh=mesh,
        scratch_types=[
            pltpu.VMEM((NCHUNK, C), jnp.int32),
            pltpu.VMEM((NCHUNK, C), jnp.int32),
            pltpu.VMEM((C, DH), jnp.float32),
            pltpu.VMEM((C, DH), jnp.float32),
            pltpu.VMEM((ZB, DH), jnp.float32),
            pltpu.VMEM_SHARED((N, DH), jnp.float32),
            pltpu.SemaphoreType.DMA,
            pltpu.SemaphoreType.DMA,
        ],
        compiler_params=pltpu.CompilerParams(use_tc_tiling_on_sc=False),
    )
    def msg_kernel(val_hbm, src_hbm, dst_hbm, out_hbm, srcb, dstb, rows_a,
                   rows_b, zb, accum, sem_a, sem_b):
        cid = lax.axis_index("c")
        sid = lax.axis_index("s")
        vals = val_hbm.at[cid]
        _zero_fill(zb, ZB, DH)
        _zero_accum(sid, zb, accum)
        plsc.subcore_barrier()
        pltpu.sync_copy(src_hbm.at[sid], srcb)
        pltpu.sync_copy(dst_hbm.at[sid], dstb)

        def gather(i, buf, sem):
            pltpu.async_copy(vals.at[srcb.at[i]], buf, sem)

        def wait(i, buf, sem):
            pltpu.make_async_copy(vals.at[srcb.at[i]], buf, sem).wait()

        def scatter(i, buf):
            pltpu.sync_copy(buf, accum.at[dstb.at[i]], add=True)

        gather(0, rows_a, sem_a)

        @pl.loop(0, NCHUNK // 2 - 1)
        def _(k):
            i = 2 * k
            gather(i + 1, rows_b, sem_b)
            wait(i, rows_a, sem_a)
            scatter(i, rows_a)
            gather(i + 2, rows_a, sem_a)
            wait(i + 1, rows_b, sem_b)
            scatter(i + 1, rows_b)

        gather(NCHUNK - 1, rows_b, sem_b)
        wait(NCHUNK - 2, rows_a, sem_a)
        scatter(NCHUNK - 2, rows_a)
        wait(NCHUNK - 1, rows_b, sem_b)
        scatter(NCHUNK - 1, rows_b)

        plsc.subcore_barrier()
        _drain_accum(cid, sid, accum, out_hbm)

    return msg_kernel(values, srcw, dstw)


_BR = 1000


def _deg_dis(deg_ref):
    deg = 1.0 + deg_ref[0, :, 0:1] + deg_ref[1, :, 0:1]
    return lax.rsqrt(deg)


def _split(o_ref, v):
    o_ref[0] = v[:, :DH]
    o_ref[1] = v[:, DH:]


def _tc_hws1(degp, x, W_in, b_in, W1):
    def body(deg_ref, x_ref, win_ref, bin_ref, w1_ref, o_ref):
        h0 = jnp.dot(x_ref[...], win_ref[...],
                     preferred_element_type=jnp.float32) + bin_ref[...]
        hw1 = jnp.dot(h0, w1_ref[...], preferred_element_type=jnp.float32)
        _split(o_ref, hw1 * _deg_dis(deg_ref))

    return pl.pallas_call(
        body,
        grid=(N // _BR,),
        in_specs=[
            pl.BlockSpec((NC, _BR, DW), lambda i: (0, i, 0)),
            pl.BlockSpec((_BR, D), lambda i: (i, 0)),
            pl.BlockSpec((D, D), lambda i: (0, 0)),
            pl.BlockSpec((1, D), lambda i: (0, 0)),
            pl.BlockSpec((D, D), lambda i: (0, 0)),
        ],
        out_specs=pl.BlockSpec((NC, _BR, DH), lambda i: (0, i, 0)),
        out_shape=jax.ShapeDtypeStruct((NC, N, DH), jnp.float32),
    )(degp, x, W_in, b_in.reshape(1, D), W1)


def _tc_mid(degp, mp, hws1, b1, W2):
    def body(deg_ref, mp_ref, hws1_ref, b1_ref, w2_ref, o_ref):
        dis = _deg_dis(deg_ref)
        s = jnp.concatenate([mp_ref[0] + hws1_ref[0],
                             mp_ref[1] + hws1_ref[1]], axis=1)
        h1 = jnp.maximum(dis * s + b1_ref[...], 0.0)
        hw2 = jnp.dot(h1, w2_ref[...], preferred_element_type=jnp.float32)
        _split(o_ref, hw2 * dis)

    return pl.pallas_call(
        body,
        grid=(N // _BR,),
        in_specs=[
            pl.BlockSpec((NC, _BR, DW), lambda i: (0, i, 0)),
            pl.BlockSpec((NC, _BR, DH), lambda i: (0, i, 0)),
            pl.BlockSpec((NC, _BR, DH), lambda i: (0, i, 0)),
            pl.BlockSpec((1, D), lambda i: (0, 0)),
            pl.BlockSpec((D, D), lambda i: (0, 0)),
        ],
        out_specs=pl.BlockSpec((NC, _BR, DH), lambda i: (0, i, 0)),
        out_shape=jax.ShapeDtypeStruct((NC, N, DH), jnp.float32),
    )(degp, mp, hws1, b1.reshape(1, D), W2)


def _tc_final(degp, mp, hws2, b2):
    def body(deg_ref, mp_ref, hws2_ref, b2_ref, o_ref):
        dis = _deg_dis(deg_ref)
        s = jnp.concatenate([mp_ref[0] + hws2_ref[0],
                             mp_ref[1] + hws2_ref[1]], axis=1)
        o_ref[...] = dis * s + b2_ref[...]

    return pl.pallas_call(
        body,
        grid=(N // _BR,),
        in_specs=[
            pl.BlockSpec((NC, _BR, DW), lambda i: (0, i, 0)),
            pl.BlockSpec((NC, _BR, DH), lambda i: (0, i, 0)),
            pl.BlockSpec((NC, _BR, DH), lambda i: (0, i, 0)),
            pl.BlockSpec((1, D), lambda i: (0, 0)),
        ],
        out_specs=pl.BlockSpec((_BR, D), lambda i: (i, 0)),
        out_shape=jax.ShapeDtypeStruct((N, D), jnp.float32),
    )(degp, mp, hws2, b2.reshape(1, D))


def kernel(x, edge_index, W_in, b_in, W1, b1, W2, b2):
    ei = edge_index.astype(jnp.int32)
    srcw = ei[0].reshape(NS, NCHUNK, C)
    dstw = ei[1].reshape(NS, NCHUNK, C)
    degp = _sc_degree(dstw)
    hws1 = _tc_hws1(degp, x, W_in, b_in, W1)
    mp1 = _sc_message(hws1, srcw, dstw)
    hws2 = _tc_mid(degp, mp1, hws1, b1, W2)
    mp2 = _sc_message(hws2, srcw, dstw)
    return _tc_final(degp, mp2, hws2, b2)


# speedup vs baseline: 32.6046x; 1.2591x over previous
import functools

import jax
import jax.numpy as jnp
from jax import lax
from jax.experimental import pallas as pl
from jax.experimental.pallas import tpu as pltpu
from jax.experimental.pallas import tpu_sc as plsc

N = 10000
D = 128
DH = D // 2
E = 320000
NC = 2
NS = 16
EPS = E // NS
C = 125
NCHUNK = EPS // C
NCHD = NCHUNK // NC
RPT = 624
TAIL = N - NS * RPT
ZB = 48
DW = 16


def _zero_fill(buf, nrows, width):
    @pl.loop(0, nrows)
    def _(r):
        @pl.loop(0, width, step=16)
        def _(j):
            buf[r, pl.ds(j, 16)] = jnp.zeros((16,), jnp.float32)


def _zero_accum(sid, zb, accum, sem):
    @pl.loop(0, RPT // ZB)
    def _(k):
        pltpu.async_copy(zb, accum.at[pl.ds(sid * RPT + k * ZB, ZB)], sem)

    @pl.when(sid == NS - 1)
    def _():
        pltpu.async_copy(zb.at[pl.ds(0, TAIL)],
                         accum.at[pl.ds(NS * RPT, TAIL)], sem)

    @pl.loop(0, RPT // ZB)
    def _(k):
        pltpu.make_async_copy(
            zb, accum.at[pl.ds(sid * RPT + k * ZB, ZB)], sem).wait()

    @pl.when(sid == NS - 1)
    def _():
        pltpu.make_async_copy(
            zb.at[pl.ds(0, TAIL)], accum.at[pl.ds(NS * RPT, TAIL)],
            sem).wait()


def _drain_accum(cid, sid, accum, out_hbm):
    pltpu.sync_copy(accum.at[pl.ds(sid * RPT, RPT)],
                    out_hbm.at[cid, pl.ds(sid * RPT, RPT)])

    @pl.when(sid == NS - 1)
    def _():
        pltpu.sync_copy(accum.at[pl.ds(NS * RPT, TAIL)],
                        out_hbm.at[cid, pl.ds(NS * RPT, TAIL)])


def _sc_degree(dstw):
    mesh = plsc.VectorSubcoreMesh(core_axis_name="c", subcore_axis_name="s")

    @functools.partial(
        pl.kernel,
        out_type=jax.ShapeDtypeStruct((NC, N, DW), jnp.float32),
        mesh=mesh,
        scratch_types=[
            pltpu.VMEM((NCHD, C), jnp.int32),
            pltpu.VMEM((C, DW), jnp.float32),
            pltpu.VMEM((ZB, DW), jnp.float32),
            pltpu.VMEM_SHARED((N, DW), jnp.float32),
            pltpu.SemaphoreType.DMA,
            pltpu.SemaphoreType.DMA,
            pltpu.SemaphoreType.DMA,
        ],
    )
    def deg_kernel(dst_hbm, out_hbm, idxb, onesb, zb, accum, sem_z, sem_i,
                   sem_s):
        cid = lax.axis_index("c")
        sid = lax.axis_index("s")
        idx_src = dst_hbm.at[sid, pl.ds(cid * NCHD, NCHD)]
        pltpu.async_copy(idx_src, idxb, sem_i)
        _zero_fill(zb, ZB, DW)

        @pl.loop(0, C)
        def _(r):
            onesb[r, :] = jnp.ones((16,), jnp.float32)

        _zero_accum(sid, zb, accum, sem_z)
        plsc.subcore_barrier()
        pltpu.make_async_copy(idx_src, idxb, sem_i).wait()

        KF = 16

        @pl.loop(0, NCHD // KF)
        def _(g):
            @pl.loop(0, KF)
            def _(j):
                pltpu.async_copy(onesb, accum.at[idxb.at[g * KF + j]],
                                 sem_s, add=True)

            @pl.loop(0, KF)
            def _(j):
                pltpu.make_async_copy(onesb, accum.at[idxb.at[g * KF + j]],
                                      sem_s).wait()

        plsc.subcore_barrier()
        _drain_accum(cid, sid, accum, out_hbm)

    return deg_kernel(dstw)


def _sc_message(values, srcw, dstw):
    mesh = plsc.VectorSubcoreMesh(core_axis_name="c", subcore_axis_name="s")

    @functools.partial(
        pl.kernel,
        out_type=jax.ShapeDtypeStruct((NC, N, DH), jnp.float32),
        mesh=mesh,
        scratch_types=[
            pltpu.VMEM((NCHUNK, C), jnp.int32),
            pltpu.VMEM((NCHUNK, C), jnp.int32),
            pltpu.VMEM((C, DH), jnp.float32),
            pltpu.VMEM((C, DH), jnp.float32),
            pltpu.VMEM((C, DH), jnp.float32),
            pltpu.VMEM((C, DH), jnp.float32),
            pltpu.VMEM((ZB, DH), jnp.float32),
            pltpu.VMEM_SHARED((N, DH), jnp.float32),
            pltpu.SemaphoreType.DMA,
            pltpu.SemaphoreType.DMA,
            pltpu.SemaphoreType.DMA,
            pltpu.SemaphoreType.DMA,
            pltpu.SemaphoreType.DMA,
            pltpu.SemaphoreType.DMA,
        ],
        compiler_params=pltpu.CompilerParams(use_tc_tiling_on_sc=False),
    )
    def msg_kernel(val_hbm, src_hbm, dst_hbm, out_hbm, srcb, dstb, rows0,
                   rows1, rows2, rows3, zb, accum, sem_z, sem_i, sg0, sg1,
                   sg2, sg3):
        cid = lax.axis_index("c")
        sid = lax.axis_index("s")
        vals = val_hbm.at[cid]
        bufs = (rows0, rows1, rows2, rows3)
        sems = (sg0, sg1, sg2, sg3)
        pltpu.async_copy(src_hbm.at[sid], srcb, sem_i)
        pltpu.async_copy(dst_hbm.at[sid], dstb, sem_i)
        _zero_fill(zb, ZB, DH)
        _zero_accum(sid, zb, accum, sem_z)
        plsc.subcore_barrier()
        pltpu.make_async_copy(src_hbm.at[sid], srcb, sem_i).wait()
        pltpu.make_async_copy(dst_hbm.at[sid], dstb, sem_i).wait()

        def gather(i, j):
            pltpu.async_copy(vals.at[srcb.at[i]], bufs[j], sems[j])

        def wait(i, j):
            pltpu.make_async_copy(vals.at[srcb.at[i]], bufs[j], sems[j]).wait()

        def scatter(i, j):
            pltpu.sync_copy(bufs[j], accum.at[dstb.at[i]], add=True)

        gather(0, 0)
        gather(1, 1)
        gather(2, 2)

        @pl.loop(0, NCHUNK // 4 - 1)
        def _(g):
            i0 = 4 * g
            for j in range(4):
                i = i0 + j
                gather(i + 3, (j + 3) % 4)
                wait(i, j)
                scatter(i, j)

        base = NCHUNK - 4
        gather(NCHUNK - 1, 3)
        for j in range(4):
            wait(base + j, j)
            scatter(base + j, j)

        plsc.subcore_barrier()
        _drain_accum(cid, sid, accum, out_hbm)

    return msg_kernel(values, srcw, dstw)


_BR = 1000


def _deg_dis(deg_ref):
    deg = 1.0 + deg_ref[0, :, 0:1] + deg_ref[1, :, 0:1]
    return lax.rsqrt(deg)


def _split(o_ref, v):
    o_ref[0] = v[:, :DH]
    o_ref[1] = v[:, DH:]


def _tc_hws1(degp, x, W_in, b_in, W1):
    def body(deg_ref, x_ref, win_ref, bin_ref, w1_ref, o_ref):
        h0 = jnp.dot(x_ref[...], win_ref[...],
                     preferred_element_type=jnp.float32) + bin_ref[...]
        hw1 = jnp.dot(h0, w1_ref[...], preferred_element_type=jnp.float32)
        _split(o_ref, hw1 * _deg_dis(deg_ref))

    return pl.pallas_call(
        body,
        grid=(N // _BR,),
        in_specs=[
            pl.BlockSpec((NC, _BR, DW), lambda i: (0, i, 0)),
            pl.BlockSpec((_BR, D), lambda i: (i, 0)),
            pl.BlockSpec((D, D), lambda i: (0, 0)),
            pl.BlockSpec((1, D), lambda i: (0, 0)),
            pl.BlockSpec((D, D), lambda i: (0, 0)),
        ],
        out_specs=pl.BlockSpec((NC, _BR, DH), lambda i: (0, i, 0)),
        out_shape=jax.ShapeDtypeStruct((NC, N, DH), jnp.float32),
    )(degp, x, W_in, b_in.reshape(1, D), W1)


def _tc_mid(degp, mp, hws1, b1, W2):
    def body(deg_ref, mp_ref, hws1_ref, b1_ref, w2_ref, o_ref):
        dis = _deg_dis(deg_ref)
        s = jnp.concatenate([mp_ref[0] + hws1_ref[0],
                             mp_ref[1] + hws1_ref[1]], axis=1)
        h1 = jnp.maximum(dis * s + b1_ref[...], 0.0)
        hw2 = jnp.dot(h1, w2_ref[...], preferred_element_type=jnp.float32)
        _split(o_ref, hw2 * dis)

    return pl.pallas_call(
        body,
        grid=(N // _BR,),
        in_specs=[
            pl.BlockSpec((NC, _BR, DW), lambda i: (0, i, 0)),
            pl.BlockSpec((NC, _BR, DH), lambda i: (0, i, 0)),
            pl.BlockSpec((NC, _BR, DH), lambda i: (0, i, 0)),
            pl.BlockSpec((1, D), lambda i: (0, 0)),
            pl.BlockSpec((D, D), lambda i: (0, 0)),
        ],
        out_specs=pl.BlockSpec((NC, _BR, DH), lambda i: (0, i, 0)),
        out_shape=jax.ShapeDtypeStruct((NC, N, DH), jnp.float32),
    )(degp, mp, hws1, b1.reshape(1, D), W2)


def _tc_final(degp, mp, hws2, b2):
    def body(deg_ref, mp_ref, hws2_ref, b2_ref, o_ref):
        dis = _deg_dis(deg_ref)
        s = jnp.concatenate([mp_ref[0] + hws2_ref[0],
                             mp_ref[1] + hws2_ref[1]], axis=1)
        o_ref[...] = dis * s + b2_ref[...]

    return pl.pallas_call(
        body,
        grid=(N // _BR,),
        in_specs=[
            pl.BlockSpec((NC, _BR, DW), lambda i: (0, i, 0)),
            pl.BlockSpec((NC, _BR, DH), lambda i: (0, i, 0)),
            pl.BlockSpec((NC, _BR, DH), lambda i: (0, i, 0)),
            pl.BlockSpec((1, D), lambda i: (0, 0)),
        ],
        out_specs=pl.BlockSpec((_BR, D), lambda i: (i, 0)),
        out_shape=jax.ShapeDtypeStruct((N, D), jnp.float32),
    )(degp, mp, hws2, b2.reshape(1, D))


def kernel(x, edge_index, W_in, b_in, W1, b1, W2, b2):
    ei = edge_index.astype(jnp.int32)
    srcw = ei[0].reshape(NS, NCHUNK, C)
    dstw = ei[1].reshape(NS, NCHUNK, C)
    degp = _sc_degree(dstw)
    hws1 = _tc_hws1(degp, x, W_in, b_in, W1)
    mp1 = _sc_message(hws1, srcw, dstw)
    hws2 = _tc_mid(degp, mp1, hws1, b1, W2)
    mp2 = _sc_message(hws2, srcw, dstw)
    return _tc_final(degp, mp2, hws2, b2)


# speedup vs baseline: 33.6189x; 1.0311x over previous
import functools

import jax
import jax.numpy as jnp
from jax import lax
from jax.experimental import pallas as pl
from jax.experimental.pallas import tpu as pltpu
from jax.experimental.pallas import tpu_sc as plsc

N = 10000
D = 128
DH = D // 2
E = 320000
NC = 2
NS = 16
EPS = E // NS
C = 125
NCHUNK = EPS // C
NCHD = NCHUNK // NC
RPT = 624
TAIL = N - NS * RPT
ZB = 48
DW = 16


def _zero_fill(buf, nrows, width):
    @pl.loop(0, nrows)
    def _(r):
        @pl.loop(0, width, step=16)
        def _(j):
            buf[r, pl.ds(j, 16)] = jnp.zeros((16,), jnp.float32)


def _zero_accum(sid, zb, accum, sem):
    @pl.loop(0, RPT // ZB)
    def _(k):
        pltpu.async_copy(zb, accum.at[pl.ds(sid * RPT + k * ZB, ZB)], sem)

    @pl.when(sid == NS - 1)
    def _():
        pltpu.async_copy(zb.at[pl.ds(0, TAIL)],
                         accum.at[pl.ds(NS * RPT, TAIL)], sem)

    @pl.loop(0, RPT // ZB)
    def _(k):
        pltpu.make_async_copy(
            zb, accum.at[pl.ds(sid * RPT + k * ZB, ZB)], sem).wait()

    @pl.when(sid == NS - 1)
    def _():
        pltpu.make_async_copy(
            zb.at[pl.ds(0, TAIL)], accum.at[pl.ds(NS * RPT, TAIL)],
            sem).wait()


def _drain_accum(cid, sid, accum, out_hbm):
    pltpu.sync_copy(accum.at[pl.ds(sid * RPT, RPT)],
                    out_hbm.at[cid, pl.ds(sid * RPT, RPT)])

    @pl.when(sid == NS - 1)
    def _():
        pltpu.sync_copy(accum.at[pl.ds(NS * RPT, TAIL)],
                        out_hbm.at[cid, pl.ds(NS * RPT, TAIL)])


def _sc_degree(dstw):
    mesh = plsc.VectorSubcoreMesh(core_axis_name="c", subcore_axis_name="s")

    @functools.partial(
        pl.kernel,
        out_type=jax.ShapeDtypeStruct((NC, N, DW), jnp.float32),
        mesh=mesh,
        scratch_types=[
            pltpu.VMEM((NCHD, C), jnp.int32),
            pltpu.VMEM((C, DW), jnp.float32),
            pltpu.VMEM((ZB, DW), jnp.float32),
            pltpu.VMEM_SHARED((N, DW), jnp.float32),
            pltpu.SemaphoreType.DMA,
            pltpu.SemaphoreType.DMA,
            pltpu.SemaphoreType.DMA,
        ],
    )
    def deg_kernel(edge_hbm, out_hbm, idxb, onesb, zb, accum, sem_z, sem_i,
                   sem_s):
        cid = lax.axis_index("c")
        sid = lax.axis_index("s")
        idx_src = edge_hbm.at[1, sid, pl.ds(cid * NCHD, NCHD)]
        pltpu.async_copy(idx_src, idxb, sem_i)
        _zero_fill(zb, ZB, DW)

        @pl.loop(0, C)
        def _(r):
            onesb[r, :] = jnp.ones((16,), jnp.float32)

        _zero_accum(sid, zb, accum, sem_z)
        plsc.subcore_barrier()
        pltpu.make_async_copy(idx_src, idxb, sem_i).wait()

        KF = 16

        @pl.loop(0, NCHD // KF)
        def _(g):
            @pl.loop(0, KF)
            def _(j):
                pltpu.async_copy(onesb, accum.at[idxb.at[g * KF + j]],
                                 sem_s, add=True)

            @pl.loop(0, KF)
            def _(j):
                pltpu.make_async_copy(onesb, accum.at[idxb.at[g * KF + j]],
                                      sem_s).wait()

        plsc.subcore_barrier()
        _drain_accum(cid, sid, accum, out_hbm)

    return deg_kernel(dstw)


def _sc_message(values, eiw):
    mesh = plsc.VectorSubcoreMesh(core_axis_name="c", subcore_axis_name="s")

    @functools.partial(
        pl.kernel,
        out_type=jax.ShapeDtypeStruct((NC, N, DH), jnp.float32),
        mesh=mesh,
        scratch_types=[
            pltpu.VMEM((NCHUNK, C), jnp.int32),
            pltpu.VMEM((NCHUNK, C), jnp.int32),
            pltpu.VMEM((C, DH), jnp.float32),
            pltpu.VMEM((C, DH), jnp.float32),
            pltpu.VMEM((C, DH), jnp.float32),
            pltpu.VMEM((C, DH), jnp.float32),
            pltpu.VMEM((ZB, DH), jnp.float32),
            pltpu.VMEM_SHARED((N, DH), jnp.float32),
            pltpu.SemaphoreType.DMA,
            pltpu.SemaphoreType.DMA,
            pltpu.SemaphoreType.DMA,
            pltpu.SemaphoreType.DMA,
            pltpu.SemaphoreType.DMA,
            pltpu.SemaphoreType.DMA,
        ],
        compiler_params=pltpu.CompilerParams(use_tc_tiling_on_sc=False),
    )
    def msg_kernel(val_hbm, edge_hbm, out_hbm, srcb, dstb, rows0,
                   rows1, rows2, rows3, zb, accum, sem_z, sem_i, sg0, sg1,
                   sg2, sg3):
        cid = lax.axis_index("c")
        sid = lax.axis_index("s")
        vals = val_hbm.at[cid]
        bufs = (rows0, rows1, rows2, rows3)
        sems = (sg0, sg1, sg2, sg3)
        pltpu.async_copy(edge_hbm.at[0, sid], srcb, sem_i)
        pltpu.async_copy(edge_hbm.at[1, sid], dstb, sem_i)
        _zero_fill(zb, ZB, DH)
        _zero_accum(sid, zb, accum, sem_z)
        plsc.subcore_barrier()
        pltpu.make_async_copy(edge_hbm.at[0, sid], srcb, sem_i).wait()
        pltpu.make_async_copy(edge_hbm.at[1, sid], dstb, sem_i).wait()

        def gather(i, j):
            pltpu.async_copy(vals.at[srcb.at[i]], bufs[j], sems[j])

        def wait(i, j):
            pltpu.make_async_copy(vals.at[srcb.at[i]], bufs[j], sems[j]).wait()

        def scatter(i, j):
            pltpu.sync_copy(bufs[j], accum.at[dstb.at[i]], add=True)

        gather(0, 0)
        gather(1, 1)
        gather(2, 2)

        @pl.loop(0, NCHUNK // 4 - 1)
        def _(g):
            i0 = 4 * g
            for j in range(4):
                i = i0 + j
                gather(i + 3, (j + 3) % 4)
                wait(i, j)
                scatter(i, j)

        base = NCHUNK - 4
        gather(NCHUNK - 1, 3)
        for j in range(4):
            wait(base + j, j)
            scatter(base + j, j)

        plsc.subcore_barrier()
        _drain_accum(cid, sid, accum, out_hbm)

    return msg_kernel(values, eiw)


_BR = 1000


def _deg_dis(deg_ref):
    deg = 1.0 + deg_ref[0, :, 0:1] + deg_ref[1, :, 0:1]
    return lax.rsqrt(deg)


def _split(o_ref, v):
    o_ref[0] = v[:, :DH]
    o_ref[1] = v[:, DH:]


def _tc_hws1(degp, x, W_in, b_in, W1):
    def body(deg_ref, x_ref, win_ref, bin_ref, w1_ref, o_ref):
        h0 = jnp.dot(x_ref[...], win_ref[...],
                     preferred_element_type=jnp.float32) + bin_ref[...]
        hw1 = jnp.dot(h0, w1_ref[...], preferred_element_type=jnp.float32)
        _split(o_ref, hw1 * _deg_dis(deg_ref))

    return pl.pallas_call(
        body,
        grid=(N // _BR,),
        in_specs=[
            pl.BlockSpec((NC, _BR, DW), lambda i: (0, i, 0)),
            pl.BlockSpec((_BR, D), lambda i: (i, 0)),
            pl.BlockSpec((D, D), lambda i: (0, 0)),
            pl.BlockSpec((1, D), lambda i: (0, 0)),
            pl.BlockSpec((D, D), lambda i: (0, 0)),
        ],
        out_specs=pl.BlockSpec((NC, _BR, DH), lambda i: (0, i, 0)),
        out_shape=jax.ShapeDtypeStruct((NC, N, DH), jnp.float32),
    )(degp, x, W_in, b_in.reshape(1, D), W1)


def _tc_mid(degp, mp, hws1, b1, W2):
    def body(deg_ref, mp_ref, hws1_ref, b1_ref, w2_ref, o_ref):
        dis = _deg_dis(deg_ref)
        s = jnp.concatenate([mp_ref[0] + hws1_ref[0],
                             mp_ref[1] + hws1_ref[1]], axis=1)
        h1 = jnp.maximum(dis * s + b1_ref[...], 0.0)
        hw2 = jnp.dot(h1, w2_ref[...], preferred_element_type=jnp.float32)
        _split(o_ref, hw2 * dis)

    return pl.pallas_call(
        body,
        grid=(N // _BR,),
        in_specs=[
            pl.BlockSpec((NC, _BR, DW), lambda i: (0, i, 0)),
            pl.BlockSpec((NC, _BR, DH), lambda i: (0, i, 0)),
            pl.BlockSpec((NC, _BR, DH), lambda i: (0, i, 0)),
            pl.BlockSpec((1, D), lambda i: (0, 0)),
            pl.BlockSpec((D, D), lambda i: (0, 0)),
        ],
        out_specs=pl.BlockSpec((NC, _BR, DH), lambda i: (0, i, 0)),
        out_shape=jax.ShapeDtypeStruct((NC, N, DH), jnp.float32),
    )(degp, mp, hws1, b1.reshape(1, D), W2)


def _tc_final(degp, mp, hws2, b2):
    def body(deg_ref, mp_ref, hws2_ref, b2_ref, o_ref):
        dis = _deg_dis(deg_ref)
        s = jnp.concatenate([mp_ref[0] + hws2_ref[0],
                             mp_ref[1] + hws2_ref[1]], axis=1)
        o_ref[...] = dis * s + b2_ref[...]

    return pl.pallas_call(
        body,
        grid=(N // _BR,),
        in_specs=[
            pl.BlockSpec((NC, _BR, DW), lambda i: (0, i, 0)),
            pl.BlockSpec((NC, _BR, DH), lambda i: (0, i, 0)),
            pl.BlockSpec((NC, _BR, DH), lambda i: (0, i, 0)),
            pl.BlockSpec((1, D), lambda i: (0, 0)),
        ],
        out_specs=pl.BlockSpec((_BR, D), lambda i: (i, 0)),
        out_shape=jax.ShapeDtypeStruct((N, D), jnp.float32),
    )(degp, mp, hws2, b2.reshape(1, D))


def kernel(x, edge_index, W_in, b_in, W1, b1, W2, b2):
    eiw = edge_index.astype(jnp.int32).reshape(2, NS, NCHUNK, C)
    degp = _sc_degree(eiw)
    hws1 = _tc_hws1(degp, x, W_in, b_in, W1)
    mp1 = _sc_message(hws1, eiw)
    hws2 = _tc_mid(degp, mp1, hws1, b1, W2)
    mp2 = _sc_message(hws2, eiw)
    return _tc_final(degp, mp2, hws2, b2)
